# baseline (speedup 1.0000x reference)
import functools

import jax
import jax.numpy as jnp
from jax import lax
from jax.experimental import pallas as pl
from jax.experimental.pallas import tpu as pltpu
from jax.experimental.pallas import tpu_sc as plsc

N = 10000
D = 128
E = 320000
B = 64
L = 3

NC = 2
NS = 16
HALF = N // NC
CHP = 128
ES = E // NS
NCH = -(-ES // CHP)
EPAD = NCH * CHP - ES
PADV = N
ACC_ROWS = HALF + 8
RA = 312
ZR = 8
NZ = RA // ZR

NB = 10
BN = N // NB


@functools.cache
def _sc_mesh():
    return plsc.VectorSubcoreMesh(
        core_axis_name="c", subcore_axis_name="s", num_cores=NC,
        num_subcores=NS)


def _remap_edges(dstv, srcv, c):
    is0 = c == 0

    def row(j, carry):
        for k in range(CHP // 16):
            sl = (j, pl.ds(k * 16, 16))
            v = dstv[sl]
            lo = v - HALF
            vdl = jnp.where(is0, jnp.minimum(v, HALF),
                            jnp.where(lo >= 0, lo, HALF))
            dstv[sl] = vdl
            if srcv is not None:
                srcv[sl] = jnp.where(vdl < HALF, srcv[sl], 0)
        return carry

    lax.fori_loop(0, NCH, row, 0)


def _zero_acc(acc, zrow_hbm, s):
    base = s * RA

    def zstep(j, carry):
        pltpu.sync_copy(zrow_hbm, acc.at[pl.ds(base + j * ZR, ZR)])
        return carry

    lax.fori_loop(0, NZ, zstep, 0)

    @pl.when(s == NS - 1)
    def _():
        def zstep2(j, carry):
            pltpu.sync_copy(zrow_hbm, acc.at[pl.ds(NS * RA + j * ZR, ZR)])
            return carry

        lax.fori_loop(0, (ACC_ROWS - NS * RA) // ZR, zstep2, 0)


def _copy_out(acc, out_hbm, c, s):
    base = s * RA
    pltpu.sync_copy(acc.at[pl.ds(base, RA)], out_hbm.at[c, pl.ds(base, RA)])

    @pl.when(s == NS - 1)
    def _():
        pltpu.sync_copy(acc.at[pl.ds(NS * RA, HALF - NS * RA)],
                        out_hbm.at[c, pl.ds(NS * RA, HALF - NS * RA)])


def _sc_deg_body(dst_hbm, ones_hbm, zrow_hbm, out_hbm, dstv, ones_v, dsem,
                 acc):
    c = lax.axis_index("c")
    s = lax.axis_index("s")
    pltpu.sync_copy(dst_hbm.at[s], dstv)
    pltpu.sync_copy(ones_hbm, ones_v)
    _remap_edges(dstv, None, c)
    _zero_acc(acc, zrow_hbm, s)
    plsc.subcore_barrier()

    def step(i, carry):
        for k in range(8):
            pltpu.async_copy(ones_v, acc.at[dstv.at[i * 8 + k]], dsem,
                             add=True)
        for k in range(8):
            pltpu.make_async_copy(ones_v, acc.at[dstv.at[i * 8 + k]],
                                  dsem).wait()
        return carry

    lax.fori_loop(0, NCH // 8, step, 0)
    for j in range(NCH - NCH % 8, NCH):
        pltpu.async_copy(ones_v, acc.at[dstv.at[j]], dsem, add=True)
    for j in range(NCH - NCH % 8, NCH):
        pltpu.make_async_copy(ones_v, acc.at[dstv.at[j]], dsem).wait()
    plsc.subcore_barrier()
    _copy_out(acc, out_hbm, c, s)


@functools.cache
def _deg_call():
    return pl.kernel(
        _sc_deg_body,
        out_type=jax.ShapeDtypeStruct((NC, HALF, D), jnp.float32),
        mesh=_sc_mesh(),
        scratch_types=[
            pltpu.VMEM((NCH, CHP), jnp.int32),
            pltpu.VMEM((CHP, D), jnp.float32),
            pltpu.SemaphoreType.DMA,
            pltpu.VMEM_SHARED((ACC_ROWS, D), jnp.float32),
        ],
    )


def _sc_msg_body(u_hbm, src_hbm, dst_hbm, zrow_hbm, out_hbm,
                 srcv, dstv, rows0, rows1, rows2,
                 g0, g1, g2, s0, s1, s2, acc):
    c = lax.axis_index("c")
    s = lax.axis_index("s")
    pltpu.sync_copy(src_hbm.at[s], srcv)
    pltpu.sync_copy(dst_hbm.at[s], dstv)
    _remap_edges(dstv, srcv, c)
    _zero_acc(acc, zrow_hbm, s)
    plsc.subcore_barrier()

    rows = (rows0, rows1, rows2)
    gs = (g0, g1, g2)
    ss = (s0, s1, s2)

    def gather(j, b):
        pltpu.async_copy(u_hbm.at[srcv.at[j]], rows[b], gs[b])

    def gather_wait(j, b):
        pltpu.make_async_copy(u_hbm.at[srcv.at[j]], rows[b], gs[b]).wait()

    def scat(j, b):
        pltpu.async_copy(rows[b], acc.at[dstv.at[j]], ss[b], add=True)

    def scat_wait(j, b):
        pltpu.make_async_copy(rows[b], acc.at[dstv.at[j]], ss[b]).wait()

    gather(0, 0)
    gather(1, 1)

    def outer(i, carry):
        for b in range(3):
            jj = 3 * i + b
            gather_wait(jj, b)
            scat(jj, b)

            @pl.when(jj >= 1)
            def _():
                scat_wait(jj - 1, (b + 2) % 3)

            @pl.when(jj + 2 < NCH)
            def _():
                gather(jj + 2, (b + 2) % 3)
        return carry

    lax.fori_loop(0, NCH // 3, outer, 0)
    jt = NCH - 1
    gather_wait(jt, jt % 3)
    scat(jt, jt % 3)
    for j in (NCH - 2, NCH - 1):
        scat_wait(j, j % 3)

    plsc.subcore_barrier()
    _copy_out(acc, out_hbm, c, s)


@functools.cache
def _msg_call():
    return pl.kernel(
        _sc_msg_body,
        out_type=jax.ShapeDtypeStruct((NC, HALF, D), jnp.float32),
        mesh=_sc_mesh(),
        scratch_types=[
            pltpu.VMEM((NCH, CHP), jnp.int32),
            pltpu.VMEM((NCH, CHP), jnp.int32),
            pltpu.VMEM((CHP, D), jnp.float32),
            pltpu.VMEM((CHP, D), jnp.float32),
            pltpu.VMEM((CHP, D), jnp.float32),
            pltpu.SemaphoreType.DMA,
            pltpu.SemaphoreType.DMA,
            pltpu.SemaphoreType.DMA,
            pltpu.SemaphoreType.DMA,
            pltpu.SemaphoreType.DMA,
            pltpu.SemaphoreType.DMA,
            pltpu.VMEM_SHARED((ACC_ROWS, D), jnp.float32),
        ],
    )


def _matT(a, w):
    return lax.dot_general(a, w, (((1,), (1,)), ((), ())),
                           preferred_element_type=jnp.float32)


def _silu(x):
    return x / (1.0 + jnp.exp(-x))


def _tc_dinv_body(deg_ref, dinv_ref):
    deg = deg_ref[:, 0:1] + 1.0
    dinv_ref[...] = jnp.broadcast_to(lax.rsqrt(deg), (BN, 8))


def _tc_in_body(x_ref, dinv_ref, inW_ref, inb_ref, W0_ref, h_ref, u_ref):
    dinv = dinv_ref[:, 0:1]
    h = _matT(x_ref[...], inW_ref[...]) + inb_ref[...]
    h_ref[...] = h
    u_ref[...] = _matT(h, W0_ref[...]) * dinv


def _layer_update(h_ref, u_ref, msg_ref, dinv_ref, cb_ref, g_ref, bb_ref):
    dinv = dinv_ref[:, 0:1]
    sm = (u_ref[...] + msg_ref[...]) * dinv + cb_ref[...]
    hn = _silu(sm)
    m = jnp.mean(hn, axis=-1, keepdims=True)
    v = jnp.mean((hn - m) ** 2, axis=-1, keepdims=True)
    hn = (hn - m) * lax.rsqrt(v + 1e-5) * g_ref[...] + bb_ref[...]
    return h_ref[...] + hn, dinv


def _tc_layer_body(h_ref, u_ref, msg_ref, dinv_ref, cb_ref, g_ref, bb_ref,
                   Wn_ref, ho_ref, uo_ref):
    h2, dinv = _layer_update(h_ref, u_ref, msg_ref, dinv_ref, cb_ref, g_ref,
                             bb_ref)
    ho_ref[...] = h2
    uo_ref[...] = _matT(h2, Wn_ref[...]) * dinv


def _tc_last_body(h_ref, u_ref, msg_ref, dinv_ref, cb_ref, g_ref, bb_ref,
                  ho_ref):
    h2, _ = _layer_update(h_ref, u_ref, msg_ref, dinv_ref, cb_ref, g_ref,
                          bb_ref)
    ho_ref[...] = h2


def _tc_pool_body(h_ref, b3_ref, emb_ref, sums, cnt):
    i = pl.program_id(0)

    @pl.when(i == 0)
    def _():
        sums[...] = jnp.zeros_like(sums)
        cnt[...] = jnp.zeros_like(cnt)

    bid = b3_ref[0, 0, :]
    oh = (bid[:, None] == lax.broadcasted_iota(jnp.int32, (BN, B), 1)
          ).astype(jnp.float32)
    sums[...] += lax.dot_general(oh, h_ref[...], (((0,), (0,)), ((), ())),
                                 preferred_element_type=jnp.float32)
    cnt[...] += lax.dot_general(oh, jnp.ones((BN, 8), jnp.float32),
                                (((0,), (0,)), ((), ())),
                                preferred_element_type=jnp.float32)

    @pl.when(i == NB - 1)
    def _():
        emb_ref[...] = sums[...] / jnp.maximum(cnt[...][:, 0:1], 1.0)


def _tc_head_body(h_ref, b3_ref, emb_ref, pW1_ref, pb1_ref, pW2_ref, pb2_ref,
                  A1_ref, A2_ref, hb1_ref, hW2_ref, hb2_ref, out_ref):
    h = h_ref[...]
    phys = _matT(_silu(_matT(h, pW1_ref[...]) + pb1_ref[...]),
                 pW2_ref[...]) + pb2_ref[...]
    a = h + phys
    bid = b3_ref[0, 0, :]
    oh = (bid[:, None] == lax.broadcasted_iota(jnp.int32, (BN, B), 1)
          ).astype(jnp.float32)
    emb = jnp.dot(oh, emb_ref[...], preferred_element_type=jnp.float32)
    hd = _silu(_matT(a, A1_ref[...]) + _matT(emb, A2_ref[...]) + hb1_ref[...])
    out_ref[...] = jnp.sum(hd * hW2_ref[...], axis=-1,
                           keepdims=True) + hb2_ref[0, 0]


def _row_spec(shape):
    return pl.BlockSpec(shape, lambda i: (i,) + (0,) * (len(shape) - 1))


def _full_spec(shape):
    return pl.BlockSpec(shape, lambda i: (0,) * len(shape))


_B3_SPEC = pl.BlockSpec((1, 1, BN), lambda i: (i, 0, 0))

_dinv_call = pl.pallas_call(
    _tc_dinv_body,
    grid=(NB,),
    in_specs=[_row_spec((BN, D))],
    out_specs=[_row_spec((BN, 8))],
    out_shape=[jax.ShapeDtypeStruct((N, 8), jnp.float32)],
)

_in_call = pl.pallas_call(
    _tc_in_body,
    grid=(NB,),
    in_specs=[
        _row_spec((BN, D)), _row_spec((BN, 8)), _full_spec((D, D)),
        _full_spec((1, D)), _full_spec((D, D)),
    ],
    out_specs=[_row_spec((BN, D)), _row_spec((BN, D))],
    out_shape=[jax.ShapeDtypeStruct((N, D), jnp.float32)] * 2,
)

_layer_call = pl.pallas_call(
    _tc_layer_body,
    grid=(NB,),
    in_specs=[
        _row_spec((BN, D)), _row_spec((BN, D)), _row_spec((BN, D)),
        _row_spec((BN, 8)),
        _full_spec((1, D)), _full_spec((1, D)), _full_spec((1, D)),
        _full_spec((D, D)),
    ],
    out_specs=[_row_spec((BN, D)), _row_spec((BN, D))],
    out_shape=[jax.ShapeDtypeStruct((N, D), jnp.float32)] * 2,
)

_last_call = pl.pallas_call(
    _tc_last_body,
    grid=(NB,),
    in_specs=[
        _row_spec((BN, D)), _row_spec((BN, D)), _row_spec((BN, D)),
        _row_spec((BN, 8)),
        _full_spec((1, D)), _full_spec((1, D)), _full_spec((1, D)),
    ],
    out_specs=[_row_spec((BN, D))],
    out_shape=[jax.ShapeDtypeStruct((N, D), jnp.float32)],
)

_pool_call = pl.pallas_call(
    _tc_pool_body,
    grid=(NB,),
    in_specs=[_row_spec((BN, D)), _B3_SPEC],
    out_specs=[_full_spec((B, D))],
    out_shape=[jax.ShapeDtypeStruct((B, D), jnp.float32)],
    scratch_shapes=[
        pltpu.VMEM((B, D), jnp.float32),
        pltpu.VMEM((B, 8), jnp.float32),
    ],
)

_head_call = pl.pallas_call(
    _tc_head_body,
    grid=(NB,),
    in_specs=[
        _row_spec((BN, D)), _B3_SPEC, _full_spec((B, D)),
        _full_spec((D, D)), _full_spec((1, D)),
        _full_spec((D, D)), _full_spec((1, D)),
        _full_spec((D, D)), _full_spec((D, D)), _full_spec((1, D)),
        _full_spec((1, D)), _full_spec((1, 1)),
    ],
    out_specs=[_row_spec((BN, 1))],
    out_shape=[jax.ShapeDtypeStruct((N, 1), jnp.float32)],
)


def kernel(x, edge_index, batch, in_W, in_b, conv_W, conv_b, ln_g, ln_b,
           p_W1, p_b1, p_W2, p_b2, h_W1, h_b1, h_W2, h_b2):
    f32 = jnp.float32
    i32 = jnp.int32
    src_p = jnp.concatenate(
        [edge_index[0].reshape(NS, ES),
         jnp.zeros((NS, EPAD), i32)], axis=1).reshape(NS, NCH, CHP)
    dst_p = jnp.concatenate(
        [edge_index[1].reshape(NS, ES),
         jnp.full((NS, EPAD), PADV, i32)], axis=1).reshape(NS, NCH, CHP)
    batch3 = batch.reshape(NB, 1, BN)
    ones_rows = jnp.ones((CHP, D), f32)
    zrow = jnp.zeros((ZR, D), f32)

    deg = _deg_call()(dst_p, ones_rows, zrow).reshape(N, D)
    (dinv,) = _dinv_call(deg)
    h, u = _in_call(x, dinv, in_W, in_b.reshape(1, D), conv_W[0])
    for l in range(L):
        msg = _msg_call()(u, src_p, dst_p, zrow).reshape(N, D)
        cb = conv_b[l].reshape(1, D)
        g = ln_g[l].reshape(1, D)
        bb = ln_b[l].reshape(1, D)
        if l + 1 < L:
            h, u = _layer_call(h, u, msg, dinv, cb, g, bb, conv_W[l + 1])
        else:
            (h,) = _last_call(h, u, msg, dinv, cb, g, bb)

    (emb,) = _pool_call(h, batch3)
    (scores,) = _head_call(
        h, batch3, emb, p_W1, p_b1.reshape(1, D), p_W2, p_b2.reshape(1, D),
        h_W1[:, :D], h_W1[:, D:], h_b1.reshape(1, D), h_W2,
        h_b2.reshape(1, 1))
    return scores.reshape(N)


# speedup vs baseline: 23.4057x; 23.4057x over previous
import functools

import jax
import jax.numpy as jnp
from jax import lax
from jax.experimental import pallas as pl
from jax.experimental.pallas import tpu as pltpu
from jax.experimental.pallas import tpu_sc as plsc

N = 10000
D = 128
E = 320000
B = 64
L = 3

NC = 2
NS = 16
HALF = N // NC
CHP = 128
ES = E // NS
NCH = -(-ES // CHP)
EPAD = NCH * CHP - ES
PADV = N
ACC_ROWS = HALF + 8
RA = 312
ZR = 8
NZ = RA // ZR

NB = 10
BN = N // NB


@functools.cache
def _sc_mesh():
    return plsc.VectorSubcoreMesh(
        core_axis_name="c", subcore_axis_name="s", num_cores=NC,
        num_subcores=NS)


def _remap_edges(dstv, c):
    is0 = c == 0

    def row(j, carry):
        for k in range(CHP // 16):
            sl = (j, pl.ds(k * 16, 16))
            v = dstv[sl]
            lo = v - HALF
            vdl = jnp.where(is0, jnp.minimum(v, HALF),
                            jnp.where(lo >= 0, lo, HALF))
            dstv[sl] = vdl
        return carry

    lax.fori_loop(0, NCH, row, 0)


def _zero_acc(acc, zrow_hbm, s):
    base = s * RA

    def zstep(j, carry):
        pltpu.sync_copy(zrow_hbm, acc.at[pl.ds(base + j * ZR, ZR)])
        return carry

    lax.fori_loop(0, NZ, zstep, 0)

    @pl.when(s == NS - 1)
    def _():
        def zstep2(j, carry):
            pltpu.sync_copy(zrow_hbm, acc.at[pl.ds(NS * RA + j * ZR, ZR)])
            return carry

        lax.fori_loop(0, (ACC_ROWS - NS * RA) // ZR, zstep2, 0)


def _copy_out(acc, out_hbm, c, s):
    base = s * RA
    pltpu.sync_copy(acc.at[pl.ds(base, RA)], out_hbm.at[c, pl.ds(base, RA)])

    @pl.when(s == NS - 1)
    def _():
        pltpu.sync_copy(acc.at[pl.ds(NS * RA, HALF - NS * RA)],
                        out_hbm.at[c, pl.ds(NS * RA, HALF - NS * RA)])


def _sc_deg_body(dst_hbm, ones_hbm, zrow_hbm, out_hbm, dstv, ones_v, dsem,
                 acc):
    c = lax.axis_index("c")
    s = lax.axis_index("s")
    pltpu.sync_copy(dst_hbm.at[s], dstv)
    pltpu.sync_copy(ones_hbm, ones_v)
    _remap_edges(dstv, c)
    _zero_acc(acc, zrow_hbm, s)
    plsc.subcore_barrier()

    def step(i, carry):
        for k in range(8):
            pltpu.async_copy(ones_v, acc.at[dstv.at[i * 8 + k]], dsem,
                             add=True)
        for k in range(8):
            pltpu.make_async_copy(ones_v, acc.at[dstv.at[i * 8 + k]],
                                  dsem).wait()
        return carry

    lax.fori_loop(0, NCH // 8, step, 0)
    for j in range(NCH - NCH % 8, NCH):
        pltpu.async_copy(ones_v, acc.at[dstv.at[j]], dsem, add=True)
    for j in range(NCH - NCH % 8, NCH):
        pltpu.make_async_copy(ones_v, acc.at[dstv.at[j]], dsem).wait()
    plsc.subcore_barrier()
    _copy_out(acc, out_hbm, c, s)


@functools.cache
def _deg_call():
    return pl.kernel(
        _sc_deg_body,
        out_type=jax.ShapeDtypeStruct((NC, HALF, D), jnp.float32),
        mesh=_sc_mesh(),
        scratch_types=[
            pltpu.VMEM((NCH, CHP), jnp.int32),
            pltpu.VMEM((CHP, D), jnp.float32),
            pltpu.SemaphoreType.DMA,
            pltpu.VMEM_SHARED((ACC_ROWS, D), jnp.float32),
        ],
    )


def _sc_msg_body(u_hbm, src_hbm, dst_hbm, zrow_hbm, out_hbm,
                 srcv, dstv, rows0, rows1, rows2,
                 g0, g1, g2, s0, s1, s2, acc):
    c = lax.axis_index("c")
    s = lax.axis_index("s")
    pltpu.sync_copy(src_hbm.at[s], srcv)
    pltpu.sync_copy(dst_hbm.at[s], dstv)
    _remap_edges(dstv, c)
    _zero_acc(acc, zrow_hbm, s)
    plsc.subcore_barrier()

    rows = (rows0, rows1, rows2)
    gs = (g0, g1, g2)
    ss = (s0, s1, s2)

    def gather(j, b):
        pltpu.async_copy(u_hbm.at[srcv.at[j]], rows[b], gs[b])

    def gather_wait(j, b):
        pltpu.make_async_copy(u_hbm.at[srcv.at[j]], rows[b], gs[b]).wait()

    def scat(j, b):
        pltpu.async_copy(rows[b], acc.at[dstv.at[j]], ss[b], add=True)

    def scat_wait(j, b):
        pltpu.make_async_copy(rows[b], acc.at[dstv.at[j]], ss[b]).wait()

    gather(0, 0)
    gather(1, 1)

    def outer(i, carry):
        for b in range(3):
            jj = 3 * i + b
            gather_wait(jj, b)
            scat(jj, b)

            @pl.when(jj >= 1)
            def _():
                scat_wait(jj - 1, (b + 2) % 3)

            @pl.when(jj + 2 < NCH)
            def _():
                gather(jj + 2, (b + 2) % 3)
        return carry

    lax.fori_loop(0, NCH // 3, outer, 0)
    jt = NCH - 1
    gather_wait(jt, jt % 3)
    scat(jt, jt % 3)
    for j in (NCH - 2, NCH - 1):
        scat_wait(j, j % 3)

    plsc.subcore_barrier()
    _copy_out(acc, out_hbm, c, s)


@functools.cache
def _msg_call():
    return pl.kernel(
        _sc_msg_body,
        out_type=jax.ShapeDtypeStruct((NC, HALF, D), jnp.float32),
        mesh=_sc_mesh(),
        scratch_types=[
            pltpu.VMEM((NCH, CHP), jnp.int32),
            pltpu.VMEM((NCH, CHP), jnp.int32),
            pltpu.VMEM((CHP, D), jnp.float32),
            pltpu.VMEM((CHP, D), jnp.float32),
            pltpu.VMEM((CHP, D), jnp.float32),
            pltpu.SemaphoreType.DMA,
            pltpu.SemaphoreType.DMA,
            pltpu.SemaphoreType.DMA,
            pltpu.SemaphoreType.DMA,
            pltpu.SemaphoreType.DMA,
            pltpu.SemaphoreType.DMA,
            pltpu.VMEM_SHARED((ACC_ROWS, D), jnp.float32),
        ],
    )


def _matT(a, w):
    return lax.dot_general(a, w, (((1,), (1,)), ((), ())),
                           preferred_element_type=jnp.float32)


def _silu(x):
    return x / (1.0 + jnp.exp(-x))


def _tc_dinv_body(deg_ref, dinv_ref):
    deg = deg_ref[:, 0:1] + 1.0
    dinv_ref[...] = jnp.broadcast_to(lax.rsqrt(deg), (BN, 8))


def _tc_in_body(x_ref, dinv_ref, inW_ref, inb_ref, W0_ref, h_ref, u_ref):
    dinv = dinv_ref[:, 0:1]
    h = _matT(x_ref[...], inW_ref[...]) + inb_ref[...]
    h_ref[...] = h
    u_ref[...] = _matT(h, W0_ref[...]) * dinv


def _layer_update(h_ref, u_ref, msg_ref, dinv_ref, cb_ref, g_ref, bb_ref):
    dinv = dinv_ref[:, 0:1]
    sm = (u_ref[...] + msg_ref[...]) * dinv + cb_ref[...]
    hn = _silu(sm)
    m = jnp.mean(hn, axis=-1, keepdims=True)
    v = jnp.mean((hn - m) ** 2, axis=-1, keepdims=True)
    hn = (hn - m) * lax.rsqrt(v + 1e-5) * g_ref[...] + bb_ref[...]
    return h_ref[...] + hn, dinv


def _tc_layer_body(h_ref, u_ref, msg_ref, dinv_ref, cb_ref, g_ref, bb_ref,
                   Wn_ref, ho_ref, uo_ref):
    h2, dinv = _layer_update(h_ref, u_ref, msg_ref, dinv_ref, cb_ref, g_ref,
                             bb_ref)
    ho_ref[...] = h2
    uo_ref[...] = _matT(h2, Wn_ref[...]) * dinv


def _tc_last_body(h_ref, u_ref, msg_ref, dinv_ref, cb_ref, g_ref, bb_ref,
                  ho_ref):
    h2, _ = _layer_update(h_ref, u_ref, msg_ref, dinv_ref, cb_ref, g_ref,
                          bb_ref)
    ho_ref[...] = h2


def _tc_pool_body(h_ref, b3_ref, emb_ref, sums, cnt):
    i = pl.program_id(0)

    @pl.when(i == 0)
    def _():
        sums[...] = jnp.zeros_like(sums)
        cnt[...] = jnp.zeros_like(cnt)

    bid = b3_ref[0, 0, :]
    oh = (bid[:, None] == lax.broadcasted_iota(jnp.int32, (BN, B), 1)
          ).astype(jnp.float32)
    sums[...] += lax.dot_general(oh, h_ref[...], (((0,), (0,)), ((), ())),
                                 preferred_element_type=jnp.float32)
    cnt[...] += lax.dot_general(oh, jnp.ones((BN, 8), jnp.float32),
                                (((0,), (0,)), ((), ())),
                                preferred_element_type=jnp.float32)

    @pl.when(i == NB - 1)
    def _():
        emb_ref[...] = sums[...] / jnp.maximum(cnt[...][:, 0:1], 1.0)


def _tc_head_body(h_ref, b3_ref, emb_ref, pW1_ref, pb1_ref, pW2_ref, pb2_ref,
                  A1_ref, A2_ref, hb1_ref, hW2_ref, hb2_ref, out_ref):
    h = h_ref[...]
    phys = _matT(_silu(_matT(h, pW1_ref[...]) + pb1_ref[...]),
                 pW2_ref[...]) + pb2_ref[...]
    a = h + phys
    bid = b3_ref[0, 0, :]
    oh = (bid[:, None] == lax.broadcasted_iota(jnp.int32, (BN, B), 1)
          ).astype(jnp.float32)
    emb = jnp.dot(oh, emb_ref[...], preferred_element_type=jnp.float32)
    hd = _silu(_matT(a, A1_ref[...]) + _matT(emb, A2_ref[...]) + hb1_ref[...])
    out_ref[...] = jnp.sum(hd * hW2_ref[...], axis=-1,
                           keepdims=True) + hb2_ref[0, 0]


def _row_spec(shape):
    return pl.BlockSpec(shape, lambda i: (i,) + (0,) * (len(shape) - 1))


def _full_spec(shape):
    return pl.BlockSpec(shape, lambda i: (0,) * len(shape))


_B3_SPEC = pl.BlockSpec((1, 1, BN), lambda i: (i, 0, 0))

_dinv_call = pl.pallas_call(
    _tc_dinv_body,
    grid=(NB,),
    in_specs=[_row_spec((BN, D))],
    out_specs=[_row_spec((BN, 8))],
    out_shape=[jax.ShapeDtypeStruct((N, 8), jnp.float32)],
)

_in_call = pl.pallas_call(
    _tc_in_body,
    grid=(NB,),
    in_specs=[
        _row_spec((BN, D)), _row_spec((BN, 8)), _full_spec((D, D)),
        _full_spec((1, D)), _full_spec((D, D)),
    ],
    out_specs=[_row_spec((BN, D)), _row_spec((BN, D))],
    out_shape=[jax.ShapeDtypeStruct((N, D), jnp.float32)] * 2,
)

_layer_call = pl.pallas_call(
    _tc_layer_body,
    grid=(NB,),
    in_specs=[
        _row_spec((BN, D)), _row_spec((BN, D)), _row_spec((BN, D)),
        _row_spec((BN, 8)),
        _full_spec((1, D)), _full_spec((1, D)), _full_spec((1, D)),
        _full_spec((D, D)),
    ],
    out_specs=[_row_spec((BN, D)), _row_spec((BN, D))],
    out_shape=[jax.ShapeDtypeStruct((N, D), jnp.float32)] * 2,
)

_last_call = pl.pallas_call(
    _tc_last_body,
    grid=(NB,),
    in_specs=[
        _row_spec((BN, D)), _row_spec((BN, D)), _row_spec((BN, D)),
        _row_spec((BN, 8)),
        _full_spec((1, D)), _full_spec((1, D)), _full_spec((1, D)),
    ],
    out_specs=[_row_spec((BN, D))],
    out_shape=[jax.ShapeDtypeStruct((N, D), jnp.float32)],
)

_pool_call = pl.pallas_call(
    _tc_pool_body,
    grid=(NB,),
    in_specs=[_row_spec((BN, D)), _B3_SPEC],
    out_specs=[_full_spec((B, D))],
    out_shape=[jax.ShapeDtypeStruct((B, D), jnp.float32)],
    scratch_shapes=[
        pltpu.VMEM((B, D), jnp.float32),
        pltpu.VMEM((B, 8), jnp.float32),
    ],
)

_head_call = pl.pallas_call(
    _tc_head_body,
    grid=(NB,),
    in_specs=[
        _row_spec((BN, D)), _B3_SPEC, _full_spec((B, D)),
        _full_spec((D, D)), _full_spec((1, D)),
        _full_spec((D, D)), _full_spec((1, D)),
        _full_spec((D, D)), _full_spec((D, D)), _full_spec((1, D)),
        _full_spec((1, D)), _full_spec((1, 1)),
    ],
    out_specs=[_row_spec((BN, 1))],
    out_shape=[jax.ShapeDtypeStruct((N, 1), jnp.float32)],
)


def kernel(x, edge_index, batch, in_W, in_b, conv_W, conv_b, ln_g, ln_b,
           p_W1, p_b1, p_W2, p_b2, h_W1, h_b1, h_W2, h_b2):
    f32 = jnp.float32
    i32 = jnp.int32
    src_p = jnp.concatenate(
        [edge_index[0].reshape(NS, ES),
         jnp.zeros((NS, EPAD), i32)], axis=1).reshape(NS, NCH, CHP)
    dst_p = jnp.concatenate(
        [edge_index[1].reshape(NS, ES),
         jnp.full((NS, EPAD), PADV, i32)], axis=1).reshape(NS, NCH, CHP)
    batch3 = batch.reshape(NB, 1, BN)
    ones_rows = jnp.ones((CHP, D), f32)
    zrow = jnp.zeros((ZR, D), f32)

    deg = _deg_call()(dst_p, ones_rows, zrow).reshape(N, D)
    (dinv,) = _dinv_call(deg)
    h, u = _in_call(x, dinv, in_W, in_b.reshape(1, D), conv_W[0])
    for l in range(L):
        msg = _msg_call()(u, src_p, dst_p, zrow).reshape(N, D)
        cb = conv_b[l].reshape(1, D)
        g = ln_g[l].reshape(1, D)
        bb = ln_b[l].reshape(1, D)
        if l + 1 < L:
            h, u = _layer_call(h, u, msg, dinv, cb, g, bb, conv_W[l + 1])
        else:
            (h,) = _last_call(h, u, msg, dinv, cb, g, bb)

    (emb,) = _pool_call(h, batch3)
    (scores,) = _head_call(
        h, batch3, emb, p_W1, p_b1.reshape(1, D), p_W2, p_b2.reshape(1, D),
        h_W1[:, :D], h_W1[:, D:], h_b1.reshape(1, D), h_W2,
        h_b2.reshape(1, 1))
    return scores.reshape(N)


# speedup vs baseline: 23.5288x; 1.0053x over previous
import functools

import jax
import jax.numpy as jnp
from jax import lax
from jax.experimental import pallas as pl
from jax.experimental.pallas import tpu as pltpu
from jax.experimental.pallas import tpu_sc as plsc

N = 10000
D = 128
E = 320000
B = 64
L = 3

NC = 2
NS = 16
HALF = N // NC
CHP = 128
ES = E // NS
NCH = -(-ES // CHP)
EPAD = NCH * CHP - ES
PADV = N
ACC_ROWS = HALF + 8
RA = 312
ZR = 8
NZ = RA // ZR

NB = 10
BN = N // NB


@functools.cache
def _sc_mesh():
    return plsc.VectorSubcoreMesh(
        core_axis_name="c", subcore_axis_name="s", num_cores=NC,
        num_subcores=NS)


def _remap_edges(dstv, c):
    is0 = c == 0

    def row(j, carry):
        for k in range(CHP // 16):
            sl = (j, pl.ds(k * 16, 16))
            v = dstv[sl]
            lo = v - HALF
            vdl = jnp.where(is0, jnp.minimum(v, HALF),
                            jnp.where(lo >= 0, lo, HALF))
            dstv[sl] = vdl
        return carry

    lax.fori_loop(0, NCH, row, 0)


def _zero_acc(acc, zrow_hbm, s):
    base = s * RA

    def zstep(j, carry):
        pltpu.sync_copy(zrow_hbm, acc.at[pl.ds(base + j * ZR, ZR)])
        return carry

    lax.fori_loop(0, NZ, zstep, 0)

    @pl.when(s == NS - 1)
    def _():
        def zstep2(j, carry):
            pltpu.sync_copy(zrow_hbm, acc.at[pl.ds(NS * RA + j * ZR, ZR)])
            return carry

        lax.fori_loop(0, (ACC_ROWS - NS * RA) // ZR, zstep2, 0)


def _copy_out(acc, out_hbm, c, s):
    base = s * RA
    pltpu.sync_copy(acc.at[pl.ds(base, RA)], out_hbm.at[c, pl.ds(base, RA)])

    @pl.when(s == NS - 1)
    def _():
        pltpu.sync_copy(acc.at[pl.ds(NS * RA, HALF - NS * RA)],
                        out_hbm.at[c, pl.ds(NS * RA, HALF - NS * RA)])


def _sc_deg_body(dst_hbm, ones_hbm, zrow_hbm, out_hbm, dstv, ones_v, dsem,
                 acc):
    c = lax.axis_index("c")
    s = lax.axis_index("s")
    pltpu.sync_copy(dst_hbm.at[s], dstv)
    pltpu.sync_copy(ones_hbm, ones_v)
    _remap_edges(dstv, c)
    _zero_acc(acc, zrow_hbm, s)
    plsc.subcore_barrier()

    def step(i, carry):
        for k in range(8):
            pltpu.async_copy(ones_v, acc.at[dstv.at[i * 8 + k]], dsem,
                             add=True)
        for k in range(8):
            pltpu.make_async_copy(ones_v, acc.at[dstv.at[i * 8 + k]],
                                  dsem).wait()
        return carry

    lax.fori_loop(0, NCH // 8, step, 0)
    for j in range(NCH - NCH % 8, NCH):
        pltpu.async_copy(ones_v, acc.at[dstv.at[j]], dsem, add=True)
    for j in range(NCH - NCH % 8, NCH):
        pltpu.make_async_copy(ones_v, acc.at[dstv.at[j]], dsem).wait()
    plsc.subcore_barrier()
    _copy_out(acc, out_hbm, c, s)


@functools.cache
def _deg_call():
    return pl.kernel(
        _sc_deg_body,
        out_type=jax.ShapeDtypeStruct((NC, HALF, D), jnp.float32),
        mesh=_sc_mesh(),
        scratch_types=[
            pltpu.VMEM((NCH, CHP), jnp.int32),
            pltpu.VMEM((CHP, D), jnp.float32),
            pltpu.SemaphoreType.DMA,
            pltpu.VMEM_SHARED((ACC_ROWS, D), jnp.float32),
        ],
    )


def _sc_msg_body(u_hbm, src_hbm, dst_hbm, zrow_hbm, out_hbm,
                 srcv, dstv, rows0, rows1, rows2,
                 g0, g1, g2, s0, s1, s2, acc):
    c = lax.axis_index("c")
    s = lax.axis_index("s")
    pltpu.sync_copy(src_hbm.at[s], srcv)

    rows = (rows0, rows1, rows2)
    gs = (g0, g1, g2)
    ss = (s0, s1, s2)

    def gather(j, b):
        pltpu.async_copy(u_hbm.at[srcv.at[j]], rows[b], gs[b])

    def gather_wait(j, b):
        pltpu.make_async_copy(u_hbm.at[srcv.at[j]], rows[b], gs[b]).wait()

    def scat(j, b):
        pltpu.async_copy(rows[b], acc.at[dstv.at[j]], ss[b], add=True)

    def scat_wait(j, b):
        pltpu.make_async_copy(rows[b], acc.at[dstv.at[j]], ss[b]).wait()

    gather(0, 0)
    gather(1, 1)
    pltpu.sync_copy(dst_hbm.at[s], dstv)
    _remap_edges(dstv, c)
    _zero_acc(acc, zrow_hbm, s)
    plsc.subcore_barrier()

    def outer(i, carry):
        for b in range(3):
            jj = 3 * i + b
            gather_wait(jj, b)
            scat(jj, b)

            @pl.when(jj >= 1)
            def _():
                scat_wait(jj - 1, (b + 2) % 3)

            @pl.when(jj + 2 < NCH)
            def _():
                gather(jj + 2, (b + 2) % 3)
        return carry

    lax.fori_loop(0, NCH // 3, outer, 0)
    jt = NCH - 1
    gather_wait(jt, jt % 3)
    scat(jt, jt % 3)
    for j in (NCH - 2, NCH - 1):
        scat_wait(j, j % 3)

    plsc.subcore_barrier()
    _copy_out(acc, out_hbm, c, s)


@functools.cache
def _msg_call():
    return pl.kernel(
        _sc_msg_body,
        out_type=jax.ShapeDtypeStruct((NC, HALF, D), jnp.float32),
        mesh=_sc_mesh(),
        scratch_types=[
            pltpu.VMEM((NCH, CHP), jnp.int32),
            pltpu.VMEM((NCH, CHP), jnp.int32),
            pltpu.VMEM((CHP, D), jnp.float32),
            pltpu.VMEM((CHP, D), jnp.float32),
            pltpu.VMEM((CHP, D), jnp.float32),
            pltpu.SemaphoreType.DMA,
            pltpu.SemaphoreType.DMA,
            pltpu.SemaphoreType.DMA,
            pltpu.SemaphoreType.DMA,
            pltpu.SemaphoreType.DMA,
            pltpu.SemaphoreType.DMA,
            pltpu.VMEM_SHARED((ACC_ROWS, D), jnp.float32),
        ],
    )


def _matT(a, w):
    return lax.dot_general(a, w, (((1,), (1,)), ((), ())),
                           preferred_element_type=jnp.float32)


def _silu(x):
    return x / (1.0 + jnp.exp(-x))


def _tc_h_body(x_ref, inW_ref, inb_ref, h_ref):
    h_ref[...] = _matT(x_ref[...], inW_ref[...]) + inb_ref[...]


def _tc_u0_body(h_ref, deg_ref, W0_ref, u_ref, dinv_ref):
    dinv = lax.rsqrt(deg_ref[:, 0:1] + 1.0)
    dinv_ref[...] = jnp.broadcast_to(dinv, (BN, 8))
    u_ref[...] = _matT(h_ref[...], W0_ref[...]) * dinv


def _layer_update(h_ref, u_ref, msg_ref, dinv_ref, cb_ref, g_ref, bb_ref):
    dinv = dinv_ref[:, 0:1]
    sm = (u_ref[...] + msg_ref[...]) * dinv + cb_ref[...]
    hn = _silu(sm)
    m = jnp.mean(hn, axis=-1, keepdims=True)
    v = jnp.mean((hn - m) ** 2, axis=-1, keepdims=True)
    hn = (hn - m) * lax.rsqrt(v + 1e-5) * g_ref[...] + bb_ref[...]
    return h_ref[...] + hn, dinv


def _tc_layer_body(h_ref, u_ref, msg_ref, dinv_ref, cb_ref, g_ref, bb_ref,
                   Wn_ref, ho_ref, uo_ref):
    h2, dinv = _layer_update(h_ref, u_ref, msg_ref, dinv_ref, cb_ref, g_ref,
                             bb_ref)
    ho_ref[...] = h2
    uo_ref[...] = _matT(h2, Wn_ref[...]) * dinv


def _tc_last_body(h_ref, u_ref, msg_ref, dinv_ref, cb_ref, g_ref, bb_ref,
                  ho_ref):
    h2, _ = _layer_update(h_ref, u_ref, msg_ref, dinv_ref, cb_ref, g_ref,
                          bb_ref)
    ho_ref[...] = h2


def _tc_pool_body(h_ref, b3_ref, emb_ref, sums, cnt):
    i = pl.program_id(0)

    @pl.when(i == 0)
    def _():
        sums[...] = jnp.zeros_like(sums)
        cnt[...] = jnp.zeros_like(cnt)

    bid = b3_ref[0, 0, :]
    oh = (bid[:, None] == lax.broadcasted_iota(jnp.int32, (BN, B), 1)
          ).astype(jnp.float32)
    sums[...] += lax.dot_general(oh, h_ref[...], (((0,), (0,)), ((), ())),
                                 preferred_element_type=jnp.float32)
    cnt[...] += lax.dot_general(oh, jnp.ones((BN, 8), jnp.float32),
                                (((0,), (0,)), ((), ())),
                                preferred_element_type=jnp.float32)

    @pl.when(i == NB - 1)
    def _():
        emb_ref[...] = sums[...] / jnp.maximum(cnt[...][:, 0:1], 1.0)


def _tc_head_body(h_ref, b3_ref, emb_ref, pW1_ref, pb1_ref, pW2_ref, pb2_ref,
                  A1_ref, A2_ref, hb1_ref, hW2_ref, hb2_ref, out_ref):
    h = h_ref[...]
    phys = _matT(_silu(_matT(h, pW1_ref[...]) + pb1_ref[...]),
                 pW2_ref[...]) + pb2_ref[...]
    a = h + phys
    bid = b3_ref[0, 0, :]
    oh = (bid[:, None] == lax.broadcasted_iota(jnp.int32, (BN, B), 1)
          ).astype(jnp.float32)
    emb = jnp.dot(oh, emb_ref[...], preferred_element_type=jnp.float32)
    hd = _silu(_matT(a, A1_ref[...]) + _matT(emb, A2_ref[...]) + hb1_ref[...])
    out_ref[...] = jnp.sum(hd * hW2_ref[...], axis=-1,
                           keepdims=True) + hb2_ref[0, 0]


def _row_spec(shape):
    return pl.BlockSpec(shape, lambda i: (i,) + (0,) * (len(shape) - 1))


def _full_spec(shape):
    return pl.BlockSpec(shape, lambda i: (0,) * len(shape))


_B3_SPEC = pl.BlockSpec((1, 1, BN), lambda i: (i, 0, 0))

_h_call = pl.pallas_call(
    _tc_h_body,
    grid=(NB,),
    in_specs=[
        _row_spec((BN, D)), _full_spec((D, D)), _full_spec((1, D)),
    ],
    out_specs=[_row_spec((BN, D))],
    out_shape=[jax.ShapeDtypeStruct((N, D), jnp.float32)],
)

_u0_call = pl.pallas_call(
    _tc_u0_body,
    grid=(NB,),
    in_specs=[
        _row_spec((BN, D)), _row_spec((BN, D)), _full_spec((D, D)),
    ],
    out_specs=[_row_spec((BN, D)), _row_spec((BN, 8))],
    out_shape=[jax.ShapeDtypeStruct((N, D), jnp.float32),
               jax.ShapeDtypeStruct((N, 8), jnp.float32)],
)

_layer_call = pl.pallas_call(
    _tc_layer_body,
    grid=(NB,),
    in_specs=[
        _row_spec((BN, D)), _row_spec((BN, D)), _row_spec((BN, D)),
        _row_spec((BN, 8)),
        _full_spec((1, D)), _full_spec((1, D)), _full_spec((1, D)),
        _full_spec((D, D)),
    ],
    out_specs=[_row_spec((BN, D)), _row_spec((BN, D))],
    out_shape=[jax.ShapeDtypeStruct((N, D), jnp.float32)] * 2,
)

_last_call = pl.pallas_call(
    _tc_last_body,
    grid=(NB,),
    in_specs=[
        _row_spec((BN, D)), _row_spec((BN, D)), _row_spec((BN, D)),
        _row_spec((BN, 8)),
        _full_spec((1, D)), _full_spec((1, D)), _full_spec((1, D)),
    ],
    out_specs=[_row_spec((BN, D))],
    out_shape=[jax.ShapeDtypeStruct((N, D), jnp.float32)],
)

_pool_call = pl.pallas_call(
    _tc_pool_body,
    grid=(NB,),
    in_specs=[_row_spec((BN, D)), _B3_SPEC],
    out_specs=[_full_spec((B, D))],
    out_shape=[jax.ShapeDtypeStruct((B, D), jnp.float32)],
    scratch_shapes=[
        pltpu.VMEM((B, D), jnp.float32),
        pltpu.VMEM((B, 8), jnp.float32),
    ],
)

_head_call = pl.pallas_call(
    _tc_head_body,
    grid=(NB,),
    in_specs=[
        _row_spec((BN, D)), _B3_SPEC, _full_spec((B, D)),
        _full_spec((D, D)), _full_spec((1, D)),
        _full_spec((D, D)), _full_spec((1, D)),
        _full_spec((D, D)), _full_spec((D, D)), _full_spec((1, D)),
        _full_spec((1, D)), _full_spec((1, 1)),
    ],
    out_specs=[_row_spec((BN, 1))],
    out_shape=[jax.ShapeDtypeStruct((N, 1), jnp.float32)],
)


def kernel(x, edge_index, batch, in_W, in_b, conv_W, conv_b, ln_g, ln_b,
           p_W1, p_b1, p_W2, p_b2, h_W1, h_b1, h_W2, h_b2):
    f32 = jnp.float32
    i32 = jnp.int32
    src_p = jnp.concatenate(
        [edge_index[0].reshape(NS, ES),
         jnp.zeros((NS, EPAD), i32)], axis=1).reshape(NS, NCH, CHP)
    dst_p = jnp.concatenate(
        [edge_index[1].reshape(NS, ES),
         jnp.full((NS, EPAD), PADV, i32)], axis=1).reshape(NS, NCH, CHP)
    batch3 = batch.reshape(NB, 1, BN)
    ones_rows = jnp.ones((CHP, D), f32)
    zrow = jnp.zeros((ZR, D), f32)

    deg = _deg_call()(dst_p, ones_rows, zrow).reshape(N, D)
    (h,) = _h_call(x, in_W, in_b.reshape(1, D))
    u, dinv = _u0_call(h, deg, conv_W[0])
    for l in range(L):
        msg = _msg_call()(u, src_p, dst_p, zrow).reshape(N, D)
        cb = conv_b[l].reshape(1, D)
        g = ln_g[l].reshape(1, D)
        bb = ln_b[l].reshape(1, D)
        if l + 1 < L:
            h, u = _layer_call(h, u, msg, dinv, cb, g, bb, conv_W[l + 1])
        else:
            (h,) = _last_call(h, u, msg, dinv, cb, g, bb)

    (emb,) = _pool_call(h, batch3)
    (scores,) = _head_call(
        h, batch3, emb, p_W1, p_b1.reshape(1, D), p_W2, p_b2.reshape(1, D),
        h_W1[:, :D], h_W1[:, D:], h_b1.reshape(1, D), h_W2,
        h_b2.reshape(1, 1))
    return scores.reshape(N)


# speedup vs baseline: 29.2001x; 1.2410x over previous
import functools

import jax
import jax.numpy as jnp
from jax import lax
from jax.experimental import pallas as pl
from jax.experimental.pallas import tpu as pltpu
from jax.experimental.pallas import tpu_sc as plsc

N = 10000
D = 128
E = 320000
B = 64
L = 3

NC = 2
NS = 16
HALF = N // NC
CHP = 128
ES = E // NS
NCH = -(-ES // CHP)
EPAD = NCH * CHP - ES
PADV = N
ACC_ROWS = HALF + 8
RA = 312
ZR = 8
NZ = RA // ZR

NB = 10
BN = N // NB


@functools.cache
def _sc_mesh():
    return plsc.VectorSubcoreMesh(
        core_axis_name="c", subcore_axis_name="s", num_cores=NC,
        num_subcores=NS)


def _remap_edges(dstv, c):
    is0 = c == 0
    iota = lax.broadcasted_iota(jnp.int32, (16,), 0)
    trash = HALF + (iota & 7)

    def row(j, carry):
        for k in range(CHP // 16):
            sl = (j, pl.ds(k * 16, 16))
            v = dstv[sl]
            lo = v - HALF
            owned = jnp.where(is0, v < HALF, (lo >= 0) & (lo < HALF))
            local = jnp.where(is0, v, lo)
            dstv[sl] = jnp.where(owned, local, trash)
        return carry

    lax.fori_loop(0, NCH, row, 0)


def _zero_acc(acc, zrow_hbm, s):
    base = s * RA

    def zstep(j, carry):
        pltpu.sync_copy(zrow_hbm, acc.at[pl.ds(base + j * ZR, ZR)])
        return carry

    lax.fori_loop(0, NZ, zstep, 0)

    @pl.when(s == NS - 1)
    def _():
        def zstep2(j, carry):
            pltpu.sync_copy(zrow_hbm, acc.at[pl.ds(NS * RA + j * ZR, ZR)])
            return carry

        lax.fori_loop(0, (ACC_ROWS - NS * RA) // ZR, zstep2, 0)


def _copy_out(acc, out_hbm, c, s):
    base = s * RA
    pltpu.sync_copy(acc.at[pl.ds(base, RA)], out_hbm.at[c, pl.ds(base, RA)])

    @pl.when(s == NS - 1)
    def _():
        pltpu.sync_copy(acc.at[pl.ds(NS * RA, HALF - NS * RA)],
                        out_hbm.at[c, pl.ds(NS * RA, HALF - NS * RA)])


def _sc_deg_body(dst_hbm, ones_hbm, zrow_hbm, out_hbm, dstv, ones_v, dsem,
                 acc):
    c = lax.axis_index("c")
    s = lax.axis_index("s")
    pltpu.sync_copy(dst_hbm.at[s], dstv)
    pltpu.sync_copy(ones_hbm, ones_v)
    _remap_edges(dstv, c)
    _zero_acc(acc, zrow_hbm, s)
    plsc.subcore_barrier()

    def step(i, carry):
        for k in range(8):
            pltpu.async_copy(ones_v, acc.at[dstv.at[i * 8 + k]], dsem,
                             add=True)
        for k in range(8):
            pltpu.make_async_copy(ones_v, acc.at[dstv.at[i * 8 + k]],
                                  dsem).wait()
        return carry

    lax.fori_loop(0, NCH // 8, step, 0)
    for j in range(NCH - NCH % 8, NCH):
        pltpu.async_copy(ones_v, acc.at[dstv.at[j]], dsem, add=True)
    for j in range(NCH - NCH % 8, NCH):
        pltpu.make_async_copy(ones_v, acc.at[dstv.at[j]], dsem).wait()
    plsc.subcore_barrier()
    _copy_out(acc, out_hbm, c, s)


@functools.cache
def _deg_call():
    return pl.kernel(
        _sc_deg_body,
        out_type=jax.ShapeDtypeStruct((NC, HALF, D), jnp.float32),
        mesh=_sc_mesh(),
        scratch_types=[
            pltpu.VMEM((NCH, CHP), jnp.int32),
            pltpu.VMEM((CHP, D), jnp.float32),
            pltpu.SemaphoreType.DMA,
            pltpu.VMEM_SHARED((ACC_ROWS, D), jnp.float32),
        ],
    )


def _sc_msg_body(u_hbm, src_hbm, dst_hbm, zrow_hbm, out_hbm,
                 srcv, dstv, rows0, rows1, rows2,
                 g0, g1, g2, s0, s1, s2, acc):
    c = lax.axis_index("c")
    s = lax.axis_index("s")
    pltpu.sync_copy(src_hbm.at[s], srcv)

    rows = (rows0, rows1, rows2)
    gs = (g0, g1, g2)
    ss = (s0, s1, s2)

    def gather(j, b):
        pltpu.async_copy(u_hbm.at[srcv.at[j]], rows[b], gs[b])

    def gather_wait(j, b):
        pltpu.make_async_copy(u_hbm.at[srcv.at[j]], rows[b], gs[b]).wait()

    def scat(j, b):
        pltpu.async_copy(rows[b], acc.at[dstv.at[j]], ss[b], add=True)

    def scat_wait(j, b):
        pltpu.make_async_copy(rows[b], acc.at[dstv.at[j]], ss[b]).wait()

    gather(0, 0)
    gather(1, 1)
    pltpu.sync_copy(dst_hbm.at[s], dstv)
    _remap_edges(dstv, c)
    _zero_acc(acc, zrow_hbm, s)
    plsc.subcore_barrier()

    def outer(i, carry):
        for b in range(3):
            jj = 3 * i + b
            gather_wait(jj, b)
            scat(jj, b)

            @pl.when(jj >= 1)
            def _():
                scat_wait(jj - 1, (b + 2) % 3)

            @pl.when(jj + 2 < NCH)
            def _():
                gather(jj + 2, (b + 2) % 3)
        return carry

    lax.fori_loop(0, NCH // 3, outer, 0)
    jt = NCH - 1
    gather_wait(jt, jt % 3)
    scat(jt, jt % 3)
    for j in (NCH - 2, NCH - 1):
        scat_wait(j, j % 3)

    plsc.subcore_barrier()
    _copy_out(acc, out_hbm, c, s)


@functools.cache
def _msg_call():
    return pl.kernel(
        _sc_msg_body,
        out_type=jax.ShapeDtypeStruct((NC, HALF, D), jnp.float32),
        mesh=_sc_mesh(),
        scratch_types=[
            pltpu.VMEM((NCH, CHP), jnp.int32),
            pltpu.VMEM((NCH, CHP), jnp.int32),
            pltpu.VMEM((CHP, D), jnp.float32),
            pltpu.VMEM((CHP, D), jnp.float32),
            pltpu.VMEM((CHP, D), jnp.float32),
            pltpu.SemaphoreType.DMA,
            pltpu.SemaphoreType.DMA,
            pltpu.SemaphoreType.DMA,
            pltpu.SemaphoreType.DMA,
            pltpu.SemaphoreType.DMA,
            pltpu.SemaphoreType.DMA,
            pltpu.VMEM_SHARED((ACC_ROWS, D), jnp.float32),
        ],
    )


def _matT(a, w):
    return lax.dot_general(a, w, (((1,), (1,)), ((), ())),
                           preferred_element_type=jnp.float32)


def _silu(x):
    return x / (1.0 + jnp.exp(-x))


def _tc_h_body(x_ref, inW_ref, inb_ref, h_ref):
    h_ref[...] = _matT(x_ref[...], inW_ref[...]) + inb_ref[...]


def _tc_u0_body(h_ref, deg_ref, W0_ref, u_ref, dinv_ref):
    dinv = lax.rsqrt(deg_ref[:, 0:1] + 1.0)
    dinv_ref[...] = jnp.broadcast_to(dinv, (BN, 8))
    u_ref[...] = _matT(h_ref[...], W0_ref[...]) * dinv


def _layer_update(h_ref, u_ref, msg_ref, dinv_ref, cb_ref, g_ref, bb_ref):
    dinv = dinv_ref[:, 0:1]
    sm = (u_ref[...] + msg_ref[...]) * dinv + cb_ref[...]
    hn = _silu(sm)
    m = jnp.mean(hn, axis=-1, keepdims=True)
    v = jnp.mean((hn - m) ** 2, axis=-1, keepdims=True)
    hn = (hn - m) * lax.rsqrt(v + 1e-5) * g_ref[...] + bb_ref[...]
    return h_ref[...] + hn, dinv


def _tc_layer_body(h_ref, u_ref, msg_ref, dinv_ref, cb_ref, g_ref, bb_ref,
                   Wn_ref, ho_ref, uo_ref):
    h2, dinv = _layer_update(h_ref, u_ref, msg_ref, dinv_ref, cb_ref, g_ref,
                             bb_ref)
    ho_ref[...] = h2
    uo_ref[...] = _matT(h2, Wn_ref[...]) * dinv


def _tc_last_body(h_ref, u_ref, msg_ref, dinv_ref, cb_ref, g_ref, bb_ref,
                  ho_ref):
    h2, _ = _layer_update(h_ref, u_ref, msg_ref, dinv_ref, cb_ref, g_ref,
                          bb_ref)
    ho_ref[...] = h2


def _tc_pool_body(h_ref, b3_ref, emb_ref, sums, cnt):
    i = pl.program_id(0)

    @pl.when(i == 0)
    def _():
        sums[...] = jnp.zeros_like(sums)
        cnt[...] = jnp.zeros_like(cnt)

    bid = b3_ref[0, 0, :]
    oh = (bid[:, None] == lax.broadcasted_iota(jnp.int32, (BN, B), 1)
          ).astype(jnp.float32)
    sums[...] += lax.dot_general(oh, h_ref[...], (((0,), (0,)), ((), ())),
                                 preferred_element_type=jnp.float32)
    cnt[...] += lax.dot_general(oh, jnp.ones((BN, 8), jnp.float32),
                                (((0,), (0,)), ((), ())),
                                preferred_element_type=jnp.float32)

    @pl.when(i == NB - 1)
    def _():
        emb_ref[...] = sums[...] / jnp.maximum(cnt[...][:, 0:1], 1.0)


def _tc_head_body(h_ref, b3_ref, emb_ref, pW1_ref, pb1_ref, pW2_ref, pb2_ref,
                  A1_ref, A2_ref, hb1_ref, hW2_ref, hb2_ref, out_ref):
    h = h_ref[...]
    phys = _matT(_silu(_matT(h, pW1_ref[...]) + pb1_ref[...]),
                 pW2_ref[...]) + pb2_ref[...]
    a = h + phys
    bid = b3_ref[0, 0, :]
    oh = (bid[:, None] == lax.broadcasted_iota(jnp.int32, (BN, B), 1)
          ).astype(jnp.float32)
    emb = jnp.dot(oh, emb_ref[...], preferred_element_type=jnp.float32)
    hd = _silu(_matT(a, A1_ref[...]) + _matT(emb, A2_ref[...]) + hb1_ref[...])
    out_ref[...] = jnp.sum(hd * hW2_ref[...], axis=-1,
                           keepdims=True) + hb2_ref[0, 0]


def _row_spec(shape):
    return pl.BlockSpec(shape, lambda i: (i,) + (0,) * (len(shape) - 1))


def _full_spec(shape):
    return pl.BlockSpec(shape, lambda i: (0,) * len(shape))


_B3_SPEC = pl.BlockSpec((1, 1, BN), lambda i: (i, 0, 0))

_h_call = pl.pallas_call(
    _tc_h_body,
    grid=(NB,),
    in_specs=[
        _row_spec((BN, D)), _full_spec((D, D)), _full_spec((1, D)),
    ],
    out_specs=[_row_spec((BN, D))],
    out_shape=[jax.ShapeDtypeStruct((N, D), jnp.float32)],
)

_u0_call = pl.pallas_call(
    _tc_u0_body,
    grid=(NB,),
    in_specs=[
        _row_spec((BN, D)), _row_spec((BN, D)), _full_spec((D, D)),
    ],
    out_specs=[_row_spec((BN, D)), _row_spec((BN, 8))],
    out_shape=[jax.ShapeDtypeStruct((N, D), jnp.float32),
               jax.ShapeDtypeStruct((N, 8), jnp.float32)],
)

_layer_call = pl.pallas_call(
    _tc_layer_body,
    grid=(NB,),
    in_specs=[
        _row_spec((BN, D)), _row_spec((BN, D)), _row_spec((BN, D)),
        _row_spec((BN, 8)),
        _full_spec((1, D)), _full_spec((1, D)), _full_spec((1, D)),
        _full_spec((D, D)),
    ],
    out_specs=[_row_spec((BN, D)), _row_spec((BN, D))],
    out_shape=[jax.ShapeDtypeStruct((N, D), jnp.float32)] * 2,
)

_last_call = pl.pallas_call(
    _tc_last_body,
    grid=(NB,),
    in_specs=[
        _row_spec((BN, D)), _row_spec((BN, D)), _row_spec((BN, D)),
        _row_spec((BN, 8)),
        _full_spec((1, D)), _full_spec((1, D)), _full_spec((1, D)),
    ],
    out_specs=[_row_spec((BN, D))],
    out_shape=[jax.ShapeDtypeStruct((N, D), jnp.float32)],
)

_pool_call = pl.pallas_call(
    _tc_pool_body,
    grid=(NB,),
    in_specs=[_row_spec((BN, D)), _B3_SPEC],
    out_specs=[_full_spec((B, D))],
    out_shape=[jax.ShapeDtypeStruct((B, D), jnp.float32)],
    scratch_shapes=[
        pltpu.VMEM((B, D), jnp.float32),
        pltpu.VMEM((B, 8), jnp.float32),
    ],
)

_head_call = pl.pallas_call(
    _tc_head_body,
    grid=(NB,),
    in_specs=[
        _row_spec((BN, D)), _B3_SPEC, _full_spec((B, D)),
        _full_spec((D, D)), _full_spec((1, D)),
        _full_spec((D, D)), _full_spec((1, D)),
        _full_spec((D, D)), _full_spec((D, D)), _full_spec((1, D)),
        _full_spec((1, D)), _full_spec((1, 1)),
    ],
    out_specs=[_row_spec((BN, 1))],
    out_shape=[jax.ShapeDtypeStruct((N, 1), jnp.float32)],
)


def kernel(x, edge_index, batch, in_W, in_b, conv_W, conv_b, ln_g, ln_b,
           p_W1, p_b1, p_W2, p_b2, h_W1, h_b1, h_W2, h_b2):
    f32 = jnp.float32
    i32 = jnp.int32
    src_p = jnp.concatenate(
        [edge_index[0].reshape(NS, ES),
         jnp.zeros((NS, EPAD), i32)], axis=1).reshape(NS, NCH, CHP)
    dst_p = jnp.concatenate(
        [edge_index[1].reshape(NS, ES),
         jnp.full((NS, EPAD), PADV, i32)], axis=1).reshape(NS, NCH, CHP)
    batch3 = batch.reshape(NB, 1, BN)
    ones_rows = jnp.ones((CHP, D), f32)
    zrow = jnp.zeros((ZR, D), f32)

    deg = _deg_call()(dst_p, ones_rows, zrow).reshape(N, D)
    (h,) = _h_call(x, in_W, in_b.reshape(1, D))
    u, dinv = _u0_call(h, deg, conv_W[0])
    for l in range(L):
        msg = _msg_call()(u, src_p, dst_p, zrow).reshape(N, D)
        cb = conv_b[l].reshape(1, D)
        g = ln_g[l].reshape(1, D)
        bb = ln_b[l].reshape(1, D)
        if l + 1 < L:
            h, u = _layer_call(h, u, msg, dinv, cb, g, bb, conv_W[l + 1])
        else:
            (h,) = _last_call(h, u, msg, dinv, cb, g, bb)

    (emb,) = _pool_call(h, batch3)
    (scores,) = _head_call(
        h, batch3, emb, p_W1, p_b1.reshape(1, D), p_W2, p_b2.reshape(1, D),
        h_W1[:, :D], h_W1[:, D:], h_b1.reshape(1, D), h_W2,
        h_b2.reshape(1, 1))
    return scores.reshape(N)
